# baseline (speedup 1.0000x reference)
import functools

import jax
import jax.numpy as jnp
from jax import lax
from jax.experimental import pallas as pl
from jax.experimental.pallas import tpu as pltpu
from jax.experimental.pallas import tpu_sc as plsc

NUM_CLASSES = 100000
FEAT = 64
BATCH = 16384
NC = 2
NS = 16
L = 16
NW = NC * NS
B_PER_W = BATCH // NW
IDX_CHUNK = 128
N_CHUNKS = B_PER_W // IDX_CHUNK
LANE_CHUNKS = FEAT // L


def _body(feat_hbm, lab_hbm, cent_hbm, out_hbm,
          idx_v, rows_v, feat_v, shared, all_v, pvec_v, out_v, sem):
    c = lax.axis_index("c")
    s = lax.axis_index("s")
    wid = c * NS + s

    pltpu.sync_copy(lab_hbm.at[wid], idx_v)
    copies = []
    for j in range(N_CHUNKS):
        copies.append(
            pltpu.async_copy(
                cent_hbm.at[idx_v.at[j]],
                rows_v.at[pl.ds(j * IDX_CHUNK, IDX_CHUNK)],
                sem,
            )
        )
    pltpu.sync_copy(feat_hbm.at[wid], feat_v)
    for cp in copies:
        cp.wait()

    def step(i, accs):
        new = []
        for k in range(LANE_CHUNKS):
            f = feat_v[i, pl.ds(k * L, L)]
            g = rows_v[i, pl.ds(k * L, L)]
            d = f - g
            new.append(accs[k] + d * d)
        return tuple(new)

    zero = jnp.zeros((L,), jnp.float32)
    accs = lax.fori_loop(0, B_PER_W, step, (zero,) * LANE_CHUNKS)
    acc = accs[0] + accs[1] + accs[2] + accs[3]

    pvec_v[...] = acc
    pltpu.sync_copy(pvec_v, shared.at[s])
    plsc.subcore_barrier()

    @pl.when(s == 0)
    def _():
        pltpu.sync_copy(shared, all_v)

        def rstep(t, tot):
            return tot + all_v[t, pl.ds(0, L)]

        total = lax.fori_loop(0, NS, rstep, jnp.zeros((L,), jnp.float32))
        out_v[...] = total * (1.0 / (BATCH * FEAT))
        pltpu.sync_copy(out_v, out_hbm.at[c])


@jax.jit
def _center_loss(features, labels, centers):
    feat3 = features.reshape(NW, B_PER_W, FEAT)
    lab3 = labels.astype(jnp.int32).reshape(NW, N_CHUNKS, IDX_CHUNK)
    mesh = plsc.VectorSubcoreMesh(core_axis_name="c", subcore_axis_name="s")
    run = pl.kernel(
        _body,
        out_type=jax.ShapeDtypeStruct((NC, L), jnp.float32),
        mesh=mesh,
        scratch_types=[
            pltpu.VMEM((N_CHUNKS, IDX_CHUNK), jnp.int32),
            pltpu.VMEM((B_PER_W, FEAT), jnp.float32),
            pltpu.VMEM((B_PER_W, FEAT), jnp.float32),
            pltpu.VMEM_SHARED((NS, L), jnp.float32),
            pltpu.VMEM((NS, L), jnp.float32),
            pltpu.VMEM((L,), jnp.float32),
            pltpu.VMEM((L,), jnp.float32),
            pltpu.SemaphoreType.DMA,
        ],
        compiler_params=pltpu.CompilerParams(use_tc_tiling_on_sc=False),
    )
    out = run(feat3, lab3, centers)
    return jnp.sum(out)


def kernel(features, labels, centers):
    return _center_loss(features, labels, centers)


# speedup vs baseline: 1.7415x; 1.7415x over previous
import jax
import jax.numpy as jnp
from jax import lax
from jax.experimental import pallas as pl
from jax.experimental.pallas import tpu as pltpu
from jax.experimental.pallas import tpu_sc as plsc

NUM_CLASSES = 100000
FEAT = 64
BATCH = 16384
NC = 2
NS = 16
L = 16
NW = NC * NS
DIMS_PER_W = FEAT // NW
FCHUNK = 4096
NFC = BATCH // FCHUNK


def _body(featT_hbm, lab_hbm, centT_hbm, out_hbm,
          lab_v, tab_v, feat_v, shared, flat_v, pvec_v, out_v):
    c = lax.axis_index("c")
    s = lax.axis_index("s")
    wid = c * NS + s

    pltpu.sync_copy(lab_hbm, lab_v)

    def process_dim(d, acc):
        pltpu.sync_copy(centT_hbm.at[d], tab_v)
        for k in range(NFC):
            pltpu.sync_copy(featT_hbm.at[d, pl.ds(k * FCHUNK, FCHUNK)],
                            feat_v)

            def step(i, a, _k=k):
                idx = lab_v[pl.ds(_k * FCHUNK + i * L, L)]
                cv = plsc.load_gather(tab_v, [idx])
                fv = feat_v[pl.ds(i * L, L)]
                dd = fv - cv
                return a + dd * dd

            acc = lax.fori_loop(0, FCHUNK // L, step, acc)
        return acc

    acc = jnp.zeros((L,), jnp.float32)
    for t in range(DIMS_PER_W):
        acc = process_dim(wid * DIMS_PER_W + t, acc)

    pvec_v[...] = acc
    pltpu.sync_copy(pvec_v, shared.at[pl.ds(s * L, L)])
    plsc.subcore_barrier()

    @pl.when(s == 0)
    def _():
        pltpu.sync_copy(shared, flat_v)

        def rstep(t, tot):
            return tot + flat_v[pl.ds(t * L, L)]

        total = lax.fori_loop(0, NS, rstep, jnp.zeros((L,), jnp.float32))
        out_v[...] = total * (1.0 / (BATCH * FEAT))
        pltpu.sync_copy(out_v, out_hbm.at[c])


@jax.jit
def _center_loss(features, labels, centers):
    featT = features.T
    centT = centers.T
    lab = labels.astype(jnp.int32)
    mesh = plsc.VectorSubcoreMesh(core_axis_name="c", subcore_axis_name="s")
    run = pl.kernel(
        _body,
        out_type=jax.ShapeDtypeStruct((NC, L), jnp.float32),
        mesh=mesh,
        scratch_types=[
            pltpu.VMEM((BATCH,), jnp.int32),
            pltpu.VMEM((NUM_CLASSES,), jnp.float32),
            pltpu.VMEM((FCHUNK,), jnp.float32),
            pltpu.VMEM_SHARED((NS * L,), jnp.float32),
            pltpu.VMEM((NS * L,), jnp.float32),
            pltpu.VMEM((L,), jnp.float32),
            pltpu.VMEM((L,), jnp.float32),
        ],
        compiler_params=pltpu.CompilerParams(needs_layout_passes=False),
    )
    out = run(featT, lab, centT)
    return jnp.sum(out)


def kernel(features, labels, centers):
    return _center_loss(features, labels, centers)


# speedup vs baseline: 2.1803x; 1.2520x over previous
import jax
import jax.numpy as jnp
from jax import lax
from jax.experimental import pallas as pl
from jax.experimental.pallas import tpu as pltpu
from jax.experimental.pallas import tpu_sc as plsc

NUM_CLASSES = 100000
FEAT = 64
BATCH = 16384
NC = 2
NS = 16
L = 16
NW = NC * NS
DIMS_PER_W = FEAT // NW
FCHUNK = 4096
NFC = BATCH // FCHUNK


UNROLL = 4


def _body(featT_hbm, lab_hbm, centT_hbm, out_hbm,
          lab_v, tab_v, feat_v, shared, flat_v, pvec_v, out_v, sem_f):
    c = lax.axis_index("c")
    s = lax.axis_index("s")
    wid = c * NS + s

    pltpu.sync_copy(lab_hbm, lab_v)

    def fetch_feat(d, k, buf):
        return pltpu.async_copy(
            featT_hbm.at[d, pl.ds(k * FCHUNK, FCHUNK)],
            feat_v.at[pl.ds(buf * FCHUNK, FCHUNK)], sem_f)

    def process_dim(d, t, acc):
        pltpu.sync_copy(centT_hbm.at[d], tab_v)
        for k in range(NFC):
            buf = (t * NFC + k) % 2
            nxt = t * NFC + k + 1
            if nxt < DIMS_PER_W * NFC:
                cp = fetch_feat(wid * DIMS_PER_W + nxt // NFC,
                                nxt % NFC, nxt % 2)

            accs = (acc, jnp.zeros((L,), jnp.float32),
                    jnp.zeros((L,), jnp.float32),
                    jnp.zeros((L,), jnp.float32))

            @plsc.parallel_loop(0, FCHUNK // L, step=UNROLL, carry=accs)
            def accs(i, a, _k=k, _buf=buf):
                a = list(a)
                for u in range(UNROLL):
                    idx = lab_v[pl.ds(_k * FCHUNK + (i + u) * L, L)]
                    cv = plsc.load_gather(tab_v, [idx])
                    fv = feat_v[pl.ds(_buf * FCHUNK + (i + u) * L, L)]
                    df = fv - cv
                    a[u] = a[u] + df * df
                return tuple(a)

            acc = accs[0] + accs[1] + accs[2] + accs[3]
            if nxt < DIMS_PER_W * NFC:
                cp.wait()
        return acc

    first = fetch_feat(wid * DIMS_PER_W, 0, 0)
    first.wait()
    acc = jnp.zeros((L,), jnp.float32)
    for t in range(DIMS_PER_W):
        acc = process_dim(wid * DIMS_PER_W + t, t, acc)

    pvec_v[...] = acc
    pltpu.sync_copy(pvec_v, shared.at[pl.ds(s * L, L)])
    plsc.subcore_barrier()

    @pl.when(s == 0)
    def _():
        pltpu.sync_copy(shared, flat_v)

        def rstep(t, tot):
            return tot + flat_v[pl.ds(t * L, L)]

        total = lax.fori_loop(0, NS, rstep, jnp.zeros((L,), jnp.float32))
        out_v[...] = total * (1.0 / (BATCH * FEAT))
        pltpu.sync_copy(out_v, out_hbm.at[c])


@jax.jit
def _center_loss(features, labels, centers):
    featT = features.T
    centT = centers.T
    lab = labels.astype(jnp.int32)
    mesh = plsc.VectorSubcoreMesh(core_axis_name="c", subcore_axis_name="s")
    run = pl.kernel(
        _body,
        out_type=jax.ShapeDtypeStruct((NC, L), jnp.float32),
        mesh=mesh,
        scratch_types=[
            pltpu.VMEM((BATCH,), jnp.int32),
            pltpu.VMEM((NUM_CLASSES,), jnp.float32),
            pltpu.VMEM((2 * FCHUNK,), jnp.float32),
            pltpu.VMEM_SHARED((NS * L,), jnp.float32),
            pltpu.VMEM((NS * L,), jnp.float32),
            pltpu.VMEM((L,), jnp.float32),
            pltpu.VMEM((L,), jnp.float32),
            pltpu.SemaphoreType.DMA,
        ],
        compiler_params=pltpu.CompilerParams(needs_layout_passes=False),
    )
    out = run(featT, lab, centT)
    return jnp.sum(out)


def kernel(features, labels, centers):
    return _center_loss(features, labels, centers)


# speedup vs baseline: 2.1936x; 1.0061x over previous
import jax
import jax.numpy as jnp
from jax import lax
from jax.experimental import pallas as pl
from jax.experimental.pallas import tpu as pltpu
from jax.experimental.pallas import tpu_sc as plsc

NUM_CLASSES = 100000
FEAT = 64
BATCH = 16384
NC = 2
NS = 16
L = 16
NW = NC * NS
DIMS_PER_W = FEAT // NW
FCHUNK = 4096
NFC = BATCH // FCHUNK


UNROLL = 4


def _body(featT_hbm, lab_hbm, centT_hbm, out_hbm,
          lab_v, tab_v, feat_v, shared, flat_v, pvec_v, out_v, sem_f):
    c = lax.axis_index("c")
    s = lax.axis_index("s")
    wid = c * NS + s

    pltpu.sync_copy(lab_hbm, lab_v)

    def fetch_feat(d, k, buf):
        return pltpu.async_copy(
            featT_hbm.at[d, pl.ds(k * FCHUNK, FCHUNK)],
            feat_v.at[pl.ds(buf * FCHUNK, FCHUNK)], sem_f)

    def process_dim(d, t, acc):
        pltpu.sync_copy(centT_hbm.at[d], tab_v)
        for k in range(NFC):
            buf = (t * NFC + k) % 2
            nxt = t * NFC + k + 1
            if nxt < DIMS_PER_W * NFC:
                cp = fetch_feat(wid * DIMS_PER_W + nxt // NFC,
                                nxt % NFC, nxt % 2)

            accs = (acc, jnp.zeros((L,), jnp.float32),
                    jnp.zeros((L,), jnp.float32),
                    jnp.zeros((L,), jnp.float32))

            @plsc.parallel_loop(0, FCHUNK // L, step=UNROLL, carry=accs)
            def accs(i, a, _k=k, _buf=buf):
                a = list(a)
                for u in range(UNROLL):
                    idx = lab_v[pl.ds(_k * FCHUNK + (i + u) * L, L)]
                    cv = plsc.load_gather(tab_v, [idx])
                    fv = feat_v[pl.ds(_buf * FCHUNK + (i + u) * L, L)]
                    df = fv - cv
                    a[u] = a[u] + df * df
                return tuple(a)

            acc = accs[0] + accs[1] + accs[2] + accs[3]
            if nxt < DIMS_PER_W * NFC:
                cp.wait()
        return acc

    first = fetch_feat(wid * DIMS_PER_W, 0, 0)
    first.wait()
    acc = jnp.zeros((L,), jnp.float32)
    for t in range(DIMS_PER_W):
        acc = process_dim(wid * DIMS_PER_W + t, t, acc)

    pvec_v[...] = acc
    pltpu.sync_copy(pvec_v, shared.at[pl.ds(s * L, L)])
    plsc.subcore_barrier()

    @pl.when(s == 0)
    def _():
        pltpu.sync_copy(shared, flat_v)

        def rstep(t, tot):
            return tot + flat_v[pl.ds(t * L, L)]

        total = lax.fori_loop(0, NS, rstep, jnp.zeros((L,), jnp.float32))
        out_v[...] = total * (1.0 / (BATCH * FEAT))
        pltpu.sync_copy(out_v, out_hbm.at[c])


@jax.jit
def _center_loss(features, labels, centers):
    featT = features.T
    centT = centers.T
    lab = labels.astype(jnp.int32)
    mesh = plsc.VectorSubcoreMesh(core_axis_name="c", subcore_axis_name="s")
    run = pl.kernel(
        _body,
        out_type=jax.ShapeDtypeStruct((NC, L), jnp.float32),
        mesh=mesh,
        scratch_types=[
            pltpu.VMEM((BATCH,), jnp.int32),
            pltpu.VMEM((NUM_CLASSES,), jnp.float32),
            pltpu.VMEM((2 * FCHUNK,), jnp.float32),
            pltpu.VMEM_SHARED((NS * L,), jnp.float32),
            pltpu.VMEM((NS * L,), jnp.float32),
            pltpu.VMEM((L,), jnp.float32),
            pltpu.VMEM((L,), jnp.float32),
            pltpu.SemaphoreType.DMA,
        ],
        compiler_params=pltpu.CompilerParams(needs_layout_passes=False,
                                             skip_device_barrier=True),
    )
    out = run(featT, lab, centT)
    return jnp.sum(out)


def kernel(features, labels, centers):
    return _center_loss(features, labels, centers)


# speedup vs baseline: 2.4066x; 1.0971x over previous
import jax
import jax.numpy as jnp
from jax import lax
from jax.experimental import pallas as pl
from jax.experimental.pallas import tpu as pltpu
from jax.experimental.pallas import tpu_sc as plsc

NUM_CLASSES = 100000
FEAT = 64
BATCH = 16384
NC = 2
NS = 16
L = 16
NW = NC * NS
DIMS_PER_W = FEAT // NW
FCHUNK = 4096
NFC = BATCH // FCHUNK


UNROLL = 4
FRING = 3


def _body(featT_hbm, lab_hbm, centT_hbm, out_hbm,
          lab_v, tab_v, feat_v, shared, flat_v, pvec_v, out_v,
          sem_f, sem_t, sem_l):
    c = lax.axis_index("c")
    s = lax.axis_index("s")
    wid = c * NS + s
    NCHUNKS = DIMS_PER_W * NFC

    def fire_tab(d):
        return [pltpu.async_copy(centT_hbm.at[d], tab_v, sem_t)]

    def fire_feat(g):
        d = wid * DIMS_PER_W + g // NFC
        return pltpu.async_copy(
            featT_hbm.at[d, pl.ds((g % NFC) * FCHUNK, FCHUNK)],
            feat_v.at[pl.ds((g % FRING) * FCHUNK, FCHUNK)], sem_f)

    tab_cps = fire_tab(wid * DIMS_PER_W)
    feat_cps = {g: fire_feat(g) for g in range(FRING)}
    lab_cp = pltpu.async_copy(lab_hbm, lab_v, sem_l)
    lab_cp.wait()

    acc = jnp.zeros((L,), jnp.float32)
    for t in range(DIMS_PER_W):
        for cp in tab_cps:
            cp.wait()
        for k in range(NFC):
            g = t * NFC + k
            feat_cps[g].wait()
            if g + FRING < NCHUNKS:
                feat_cps[g + FRING] = fire_feat(g + FRING)

            accs = (acc, jnp.zeros((L,), jnp.float32),
                    jnp.zeros((L,), jnp.float32),
                    jnp.zeros((L,), jnp.float32))

            @plsc.parallel_loop(0, FCHUNK // L, step=UNROLL, carry=accs)
            def accs(i, a, _k=k, _buf=g % FRING):
                a = list(a)
                for u in range(UNROLL):
                    idx = lab_v[pl.ds(_k * FCHUNK + (i + u) * L, L)]
                    cv = plsc.load_gather(tab_v, [idx])
                    fv = feat_v[pl.ds(_buf * FCHUNK + (i + u) * L, L)]
                    df = fv - cv
                    a[u] = a[u] + df * df
                return tuple(a)

            acc = accs[0] + accs[1] + accs[2] + accs[3]
        if t + 1 < DIMS_PER_W:
            tab_cps = fire_tab(wid * DIMS_PER_W + t + 1)

    pvec_v[...] = acc
    pltpu.sync_copy(pvec_v, shared.at[pl.ds(s * L, L)])
    plsc.subcore_barrier()

    @pl.when(s == 0)
    def _():
        pltpu.sync_copy(shared, flat_v)

        def rstep(t, tot):
            return tot + flat_v[pl.ds(t * L, L)]

        total = lax.fori_loop(0, NS, rstep, jnp.zeros((L,), jnp.float32))
        out_v[...] = total * (1.0 / (BATCH * FEAT))
        pltpu.sync_copy(out_v, out_hbm.at[c])


@jax.jit
def _center_loss(features, labels, centers):
    featT = features.T
    centT = centers.T
    lab = labels.astype(jnp.int32)
    mesh = plsc.VectorSubcoreMesh(core_axis_name="c", subcore_axis_name="s")
    run = pl.kernel(
        _body,
        out_type=jax.ShapeDtypeStruct((NC, L), jnp.float32),
        mesh=mesh,
        scratch_types=[
            pltpu.VMEM((BATCH,), jnp.int32),
            pltpu.VMEM((NUM_CLASSES,), jnp.float32),
            pltpu.VMEM((FRING * FCHUNK,), jnp.float32),
            pltpu.VMEM_SHARED((NS * L,), jnp.float32),
            pltpu.VMEM((NS * L,), jnp.float32),
            pltpu.VMEM((L,), jnp.float32),
            pltpu.VMEM((L,), jnp.float32),
            pltpu.SemaphoreType.DMA,
            pltpu.SemaphoreType.DMA,
            pltpu.SemaphoreType.DMA,
        ],
        compiler_params=pltpu.CompilerParams(needs_layout_passes=False,
                                             skip_device_barrier=True),
    )
    out = run(featT, lab, centT)
    return jnp.sum(out)


def kernel(features, labels, centers):
    return _center_loss(features, labels, centers)


# speedup vs baseline: 2.4083x; 1.0007x over previous
import jax
import jax.numpy as jnp
from jax import lax
from jax.experimental import pallas as pl
from jax.experimental.pallas import tpu as pltpu
from jax.experimental.pallas import tpu_sc as plsc

NUM_CLASSES = 100000
FEAT = 64
BATCH = 16384
NC = 2
NS = 16
L = 16
NW = NC * NS
DIMS_PER_W = FEAT // NW
FCHUNK = 4096
NFC = BATCH // FCHUNK
UNROLL = 4
FRING = 3
NCHUNKS = DIMS_PER_W * NFC


def _body(featT_hbm, lab_hbm, centT_hbm, out_hbm,
          lab_v, tab_v, feat_v, shared, flat_v, pvec_v, out_v,
          sem_f0, sem_f1, sem_f2, sem_t, sem_l):
    c = lax.axis_index("c")
    s = lax.axis_index("s")
    wid = c * NS + s
    sem_f = (sem_f0, sem_f1, sem_f2)

    def fire_feat(g):
        d = wid * DIMS_PER_W + g // NFC
        return pltpu.async_copy(
            featT_hbm.at[d, pl.ds((g % NFC) * FCHUNK, FCHUNK)],
            feat_v.at[pl.ds((g % FRING) * FCHUNK, FCHUNK)],
            sem_f[g % FRING])

    tab_cp = pltpu.async_copy(centT_hbm.at[wid * DIMS_PER_W], tab_v, sem_t)
    feat_cps = {g: fire_feat(g) for g in range(FRING)}
    pltpu.async_copy(lab_hbm, lab_v, sem_l).wait()

    acc = jnp.zeros((L,), jnp.float32)
    for t in range(DIMS_PER_W):
        tab_cp.wait()
        for k in range(NFC):
            g = t * NFC + k
            feat_cps[g].wait()
            if g + FRING < NCHUNKS:
                feat_cps[g + FRING] = fire_feat(g + FRING)

            accs = (acc, jnp.zeros((L,), jnp.float32),
                    jnp.zeros((L,), jnp.float32),
                    jnp.zeros((L,), jnp.float32))

            @plsc.parallel_loop(0, FCHUNK // L, step=UNROLL, carry=accs)
            def accs(i, a, _k=k, _buf=g % FRING):
                a = list(a)
                for u in range(UNROLL):
                    idx = lab_v[pl.ds(_k * FCHUNK + (i + u) * L, L)]
                    cv = plsc.load_gather(tab_v, [idx])
                    fv = feat_v[pl.ds(_buf * FCHUNK + (i + u) * L, L)]
                    df = fv - cv
                    a[u] = a[u] + df * df
                return tuple(a)

            acc = accs[0] + accs[1] + accs[2] + accs[3]
        if t + 1 < DIMS_PER_W:
            tab_cp = pltpu.async_copy(centT_hbm.at[wid * DIMS_PER_W + t + 1],
                                      tab_v, sem_t)

    pvec_v[...] = acc
    pltpu.sync_copy(pvec_v, shared.at[pl.ds(s * L, L)])
    plsc.subcore_barrier()

    @pl.when(s == 0)
    def _():
        pltpu.sync_copy(shared, flat_v)

        def rstep(t, tot):
            return tot + flat_v[pl.ds(t * L, L)]

        total = lax.fori_loop(0, NS, rstep, jnp.zeros((L,), jnp.float32))
        out_v[...] = total * (1.0 / (BATCH * FEAT))
        pltpu.sync_copy(out_v, out_hbm.at[c])


@jax.jit
def _center_loss(features, labels, centers):
    featT = features.T
    centT = centers.T
    lab = labels.astype(jnp.int32)
    mesh = plsc.VectorSubcoreMesh(core_axis_name="c", subcore_axis_name="s")
    run = pl.kernel(
        _body,
        out_type=jax.ShapeDtypeStruct((NC, L), jnp.float32),
        mesh=mesh,
        scratch_types=[
            pltpu.VMEM((BATCH,), jnp.int32),
            pltpu.VMEM((NUM_CLASSES,), jnp.float32),
            pltpu.VMEM((FRING * FCHUNK,), jnp.float32),
            pltpu.VMEM_SHARED((NS * L,), jnp.float32),
            pltpu.VMEM((NS * L,), jnp.float32),
            pltpu.VMEM((L,), jnp.float32),
            pltpu.VMEM((L,), jnp.float32),
            pltpu.SemaphoreType.DMA,
            pltpu.SemaphoreType.DMA,
            pltpu.SemaphoreType.DMA,
            pltpu.SemaphoreType.DMA,
            pltpu.SemaphoreType.DMA,
        ],
        compiler_params=pltpu.CompilerParams(needs_layout_passes=False),
    )
    out = run(featT, lab, centT)
    return jnp.sum(out)


def kernel(features, labels, centers):
    return _center_loss(features, labels, centers)
